# baseline (speedup 1.0000x reference)
import functools

import jax
import jax.numpy as jnp
from jax import lax
from jax.experimental import pallas as pl
from jax.experimental.pallas import tpu as pltpu
from jax.experimental.pallas import tpu_sc as plsc

B = 16384
V = 1000000
D = 64
DP = 128
NEG = 5

NC = 2
NS = 16
L = 16
NW = NC * NS
VLIM = 999936
NQ = 245
NSUP = 16
BW_CAP = 768
BV_CAP = 4096
GW_CAP = 96
GV_CAP = 320
XT_CAP = 64
OUT_CAP = GV_CAP
DUMP_W = B
DUMP_VN = B * (NEG + 1)
NVN = B * (NEG + 1) + 1

i32 = jnp.int32
f32 = jnp.float32


def _sc_extract():
    mesh = plsc.VectorSubcoreMesh(
        core_axis_name="c", subcore_axis_name="s", num_cores=NC, num_subcores=NS
    )

    @functools.partial(
        pl.kernel,
        mesh=mesh,
        compiler_params=pltpu.CompilerParams(needs_layout_passes=False),
        out_type=[
            jax.ShapeDtypeStruct((B + 1, DP), f32),
            jax.ShapeDtypeStruct((NVN, DP), f32),
        ],
        scratch_types=[
            pltpu.VMEM((4096,), i32),
            pltpu.VMEM((BW_CAP,), i32),
            pltpu.VMEM((BW_CAP,), i32),
            pltpu.VMEM((BV_CAP,), i32),
            pltpu.VMEM((BV_CAP,), i32),
            pltpu.VMEM((NSUP * GW_CAP,), i32),
            pltpu.VMEM((NSUP * GW_CAP,), i32),
            pltpu.VMEM((NSUP * GV_CAP,), i32),
            pltpu.VMEM((NSUP * GV_CAP,), i32),
            pltpu.VMEM((16,), i32),
            pltpu.VMEM((16,), i32),
            pltpu.VMEM((XT_CAP,), i32),
            pltpu.VMEM((XT_CAP,), i32),
            pltpu.VMEM((XT_CAP,), i32),
            pltpu.VMEM((XT_CAP,), i32),
            pltpu.VMEM((128,), i32),
            pltpu.VMEM((128,), i32),
            pltpu.VMEM((64, 128), f32),
            pltpu.VMEM((64, 128), f32),
            pltpu.VMEM((64, 64), f32),
            pltpu.VMEM((64, 64), f32),
            pltpu.VMEM((OUT_CAP, DP), f32),
            pltpu.VMEM((OUT_CAP,), i32),
            pltpu.SemaphoreType.DMA,
            pltpu.SemaphoreType.DMA,
            pltpu.SemaphoreType.DMA,
        ],
    )
    def body(pos_w_hbm, pos_v_hbm, neg_hbm, wT_hbm, vT_hbm,
             tailw_hbm, tailv_hbm,
             stage_w_hbm, stage_vn_hbm,
             scan_buf, bw_i, bw_t, bv_i, bv_t, gw_i, gw_t, gv_i, gv_t,
             cnt_w, cnt_v, xw_i, xw_t, xv_i, xv_t, blk_i, blk_t,
             slab0, slab1, tw_v, tv_v, outbuf, tlist, sem0, sem1, sem_s):
        wid = lax.axis_index("s") * NC + lax.axis_index("c")
        lane = lax.iota(i32, 16)

        def zinit(u, _z):
            row = jnp.full((16,), u, i32)
            for k in range(4, 8):
                plsc.store_scatter(outbuf, [row, lane + k * L],
                                   jnp.zeros((16,), f32))
            return 0

        lax.fori_loop(0, OUT_CAP, zinit, 0)

        def scan_src(src_hbm, n_total, t_off, b_i, b_t, x_i, x_t, carry0):
            nchunk = n_total // 4096

            def chunk_fn(ci, carry):
                pltpu.sync_copy(src_hbm.at[pl.ds(ci * 4096, 4096)], scan_buf)

                def vreg_fn(u, c2):
                    cnt, xcnt = c2
                    x = scan_buf[pl.ds(u * L, L)]
                    t = ci * 4096 + u * L + lane + t_off
                    own = (((x >> 7) & 31) == wid) & (x < VLIM)
                    rank = plsc.cumsum(own.astype(i32)) - 1
                    plsc.store_scatter(b_i, [cnt + rank], x, mask=own)
                    plsc.store_scatter(b_t, [cnt + rank], t, mask=own)
                    ex = (x >= VLIM) & (wid == 0)
                    xrank = plsc.cumsum(ex.astype(i32)) - 1
                    plsc.store_scatter(x_i, [xcnt + xrank], x, mask=ex)
                    plsc.store_scatter(x_t, [xcnt + xrank], t, mask=ex)
                    return (cnt + jnp.sum(own.astype(i32)),
                            xcnt + jnp.sum(ex.astype(i32)))

                return lax.fori_loop(0, 256, vreg_fn, carry)

            return lax.fori_loop(0, nchunk, chunk_fn, carry0)

        nw_cnt, xw_cnt = scan_src(pos_w_hbm, B, 0, bw_i, bw_t, xw_i, xw_t,
                                  (jnp.int32(0), jnp.int32(0)))
        c_v = scan_src(neg_hbm, B * NEG, 0, bv_i, bv_t, xv_i, xv_t,
                       (jnp.int32(0), jnp.int32(0)))
        c_v = scan_src(pos_v_hbm, B, B * NEG, bv_i, bv_t, xv_i, xv_t, c_v)
        nv_cnt, xv_cnt = c_v

        def group(b_i, b_t, g_i, g_t, cap, gcap, n_ent, cnt_vec):
            for s in range(NSUP):
                def vfn(c, gc):
                    x = b_i[pl.ds(c * L, L)]
                    t = b_t[pl.ds(c * L, L)]
                    m = ((c * L + lane) < n_ent) & ((x >> 16) == s)
                    rank = plsc.cumsum(m.astype(i32)) - 1
                    plsc.store_scatter(g_i, [s * gcap + gc + rank], x, mask=m)
                    plsc.store_scatter(g_t, [s * gcap + gc + rank], t, mask=m)
                    return gc + jnp.sum(m.astype(i32))

                gcnt = lax.fori_loop(0, cap // L, vfn, jnp.int32(0))
                plsc.store_scatter(cnt_vec, [jnp.full((16,), s, i32)],
                                   jnp.full((16,), 1, i32) * gcnt,
                                   mask=lane == 0)

        group(bw_i, bw_t, gw_i, gw_t, BW_CAP, GW_CAP, nw_cnt, cnt_w)
        group(bv_i, bv_t, gv_i, gv_t, BV_CAP, GV_CAP, nv_cnt, cnt_v)

        slabs = (slab0, slab1)
        sems = (sem0, sem1)

        def fetch(tbl_hbm, q, r):
            m = jnp.minimum(q * 32 + wid, 7811)
            start = m * 128
            for jb in range(8):
                pltpu.async_copy(
                    tbl_hbm.at[pl.ds(jb * 8, 8), pl.ds(start, 128)],
                    slabs[r].at[pl.ds(jb * 8, 8), :], sems[r])

        def wait_slab(tbl_hbm, r):
            pltpu.make_async_copy(
                tbl_hbm.at[:, pl.ds(0, 128)], slabs[r], sems[r]).wait()

        def extract_entries(slab, n_ent, start, oc0):
            def efn(e, oc):
                iv = blk_i[pl.ds((e >> 4) * L, L)]
                tv = blk_t[pl.ds((e >> 4) * L, L)]
                sel = jnp.full((16,), e & 15, i32)
                il = jnp.take(iv, sel) - start
                row = jnp.full((16,), oc, i32)
                for k in range(4):
                    g = plsc.load_gather(slab, [lane + k * L, il])
                    plsc.store_scatter(outbuf, [row, lane + k * L], g)
                plsc.store_scatter(tlist, [jnp.full((16,), oc, i32)],
                                   jnp.take(tv, sel), mask=lane == 0)
                return oc + 1

            return lax.fori_loop(0, n_ent, efn, oc0)

        def stream_table(tbl_hbm, g_i, g_t, gcap, cnt_vec, stage_hbm, dump):
            def super_fn(s, _):
                creg = cnt_vec[pl.ds(0, 16)]
                cnt_s = jnp.take(creg, jnp.full((16,), s, i32))[0]
                nv = (cnt_s + L - 1) >> 4

                def tinit(u, _2):
                    tlist[pl.ds(u * L, L)] = jnp.full((16,), dump, i32)
                    return 0

                lax.fori_loop(0, OUT_CAP // L, tinit, 0)

                def rescan(q, oc):
                    def rfn(c, bc):
                        x = g_i[pl.ds(s * gcap + c * L, L)]
                        t = g_t[pl.ds(s * gcap + c * L, L)]
                        m = ((c * L + lane) < cnt_s) & ((x >> 12) == q)
                        rank = plsc.cumsum(m.astype(i32)) - 1
                        plsc.store_scatter(blk_i, [bc + rank], x, mask=m)
                        plsc.store_scatter(blk_t, [bc + rank], t, mask=m)
                        return bc + jnp.sum(m.astype(i32))

                    return lax.fori_loop(0, nv, rfn, jnp.int32(0))

                fetch(tbl_hbm, s * 16, 0)

                def pair_fn(h, oc):
                    for r in range(2):
                        q = s * 16 + h * 2 + r
                        fetch(tbl_hbm, q + 1, 1 - r)
                        wait_slab(tbl_hbm, r)
                        bc = rescan(q, oc)
                        mm = jnp.minimum(q * 32 + wid, 7811)
                        oc = extract_entries(slabs[r], bc, mm * 128, oc)
                    return oc

                oc = lax.fori_loop(0, 8, pair_fn, jnp.int32(0))
                wait_slab(tbl_hbm, 0)
                pltpu.async_copy(outbuf, stage_hbm.at[tlist], sem_s).wait()
                return 0

            lax.fori_loop(0, NSUP, super_fn, 0)

        stream_table(wT_hbm, gw_i, gw_t, GW_CAP, cnt_w, stage_w_hbm, DUMP_W)
        stream_table(vT_hbm, gv_i, gv_t, GV_CAP, cnt_v, stage_vn_hbm, DUMP_VN)

        @pl.when(wid == 0)
        def _():
            pltpu.sync_copy(tailw_hbm, tw_v)
            pltpu.sync_copy(tailv_hbm, tv_v)
            for (slab, x_i, x_t, xcnt, stage_hbm, dump) in (
                    (tw_v, xw_i, xw_t, xw_cnt, stage_w_hbm, DUMP_W),
                    (tv_v, xv_i, xv_t, xv_cnt, stage_vn_hbm, DUMP_VN)):
                def tinit(u, _2):
                    tlist[pl.ds(u * L, L)] = jnp.full((16,), dump, i32)
                    return 0

                lax.fori_loop(0, OUT_CAP // L, tinit, 0)

                def cpy(u, _2):
                    blk_i[pl.ds(u * L, L)] = x_i[pl.ds(u * L, L)]
                    blk_t[pl.ds(u * L, L)] = x_t[pl.ds(u * L, L)]
                    return 0

                lax.fori_loop(0, XT_CAP // L, cpy, 0)
                extract_entries(slab, xcnt, VLIM, jnp.int32(0))
                pltpu.async_copy(outbuf, stage_hbm.at[tlist], sem_s).wait()

    return body


_SC_EXTRACT = _sc_extract()


def _tc_loss_body(w_ref, v_ref, n_ref, o_ref):
    pc = pl.program_id(0)
    w = w_ref[...]
    v = v_ref[...]
    n3 = n_ref[...].reshape(B // 32, NEG, DP)
    score = jnp.sum(w * v, axis=1)
    nscore = jnp.sum(n3 * v[:, None, :], axis=2)
    lsp = jnp.minimum(score, 0.0) - jnp.log1p(jnp.exp(-jnp.abs(score)))
    m = -nscore
    lsn = jnp.minimum(m, 0.0) - jnp.log1p(jnp.exp(-jnp.abs(m)))
    part = -(jnp.sum(lsp) + jnp.sum(lsn))

    @pl.when(pc == 0)
    def _():
        o_ref[0, 0] = 0.0

    o_ref[0, 0] += part


def kernel(pos_w, pos_v, neg_v, w_emb, v_emb):
    pos_w = pos_w.astype(i32)
    pos_v = pos_v.astype(i32)
    neg_flat = neg_v.reshape(-1).astype(i32)
    wT = w_emb.T
    vT = v_emb.T
    tail_w = wT[:, VLIM:]
    tail_v = vT[:, VLIM:]

    stage_w, stage_vn = _SC_EXTRACT(pos_w, pos_v, neg_flat, wT, vT,
                                    tail_w, tail_v)

    grid = 32
    bb = B // grid
    loss = pl.pallas_call(
        _tc_loss_body,
        grid=(grid,),
        in_specs=[
            pl.BlockSpec((bb, DP), lambda c: (c, 0)),
            pl.BlockSpec((bb, DP), lambda c: (B * NEG // bb + c, 0)),
            pl.BlockSpec((bb * NEG, DP), lambda c: (c, 0)),
        ],
        out_specs=pl.BlockSpec(memory_space=pltpu.SMEM),
        out_shape=jax.ShapeDtypeStruct((1, 1), f32),
    )(stage_w, stage_vn, stage_vn)
    return loss[0, 0]


# speedup vs baseline: 7.9247x; 7.9247x over previous
import functools

import jax
import jax.numpy as jnp
from jax import lax
from jax.experimental import pallas as pl
from jax.experimental.pallas import tpu as pltpu
from jax.experimental.pallas import tpu_sc as plsc

B = 16384
V = 1000000
D = 64
NEG = 5

NC = 2
NS = 16
L = 16
NW = NC * NS
NB = B // NW
CH = 16
NCHUNK = NB // CH
KD = D // L


def _sc_scores():
    mesh = plsc.VectorSubcoreMesh(
        core_axis_name="c", subcore_axis_name="s", num_cores=NC, num_subcores=NS
    )

    @functools.partial(
        pl.kernel,
        mesh=mesh,
        compiler_params=pltpu.CompilerParams(needs_layout_passes=False),
        out_type=[
            jax.ShapeDtypeStruct((B,), jnp.float32),
            jax.ShapeDtypeStruct((B * NEG,), jnp.float32),
        ],
        scratch_types=[
            pltpu.VMEM((NB,), jnp.int32),
            pltpu.VMEM((NB,), jnp.int32),
            pltpu.VMEM((NB * NEG,), jnp.int32),
            pltpu.VMEM((NB,), jnp.int32),
            pltpu.VMEM((NB,), jnp.int32),
            pltpu.VMEM((NB * NEG,), jnp.int32),
            pltpu.VMEM((NB,), jnp.float32),
            pltpu.VMEM((NB,), jnp.float32),
            pltpu.VMEM((NB * NEG,), jnp.float32),
            pltpu.VMEM((CH, 2 * D), jnp.float32),
            pltpu.VMEM((CH, 2 * D), jnp.float32),
            pltpu.VMEM((CH * NEG, 2 * D), jnp.float32),
            pltpu.VMEM((NB,), jnp.float32),
            pltpu.VMEM((NB * NEG,), jnp.float32),
            pltpu.SemaphoreType.DMA,
            pltpu.SemaphoreType.DMA,
            pltpu.SemaphoreType.DMA,
        ],
    )
    def body(pos_w_hbm, pos_v_hbm, neg_hbm, w2_hbm, v2_hbm,
             pos_out_hbm, neg_out_hbm,
             idx_w, idx_v, idx_n, half_w, half_v, half_n,
             par_w, par_v, par_n, rows_w, rows_v, rows_n,
             pos_buf, neg_buf, sem_w, sem_v, sem_n):
        wid = lax.axis_index("s") * NC + lax.axis_index("c")
        base = wid * NB
        nbase = wid * NB * NEG

        pltpu.sync_copy(pos_w_hbm.at[pl.ds(base, NB)], idx_w)
        pltpu.sync_copy(pos_v_hbm.at[pl.ds(base, NB)], idx_v)
        pltpu.sync_copy(neg_hbm.at[pl.ds(nbase, NB * NEG)], idx_n)

        lane = lax.iota(jnp.int32, 16)

        def split(t, _):
            for raw, half, par, n_vec in ((idx_w, half_w, par_w, 2),
                                          (idx_v, half_v, par_v, 2),
                                          (idx_n, half_n, par_n, 10)):
                for u in range(n_vec):
                    o = t * n_vec * L + u * L
                    x = raw[pl.ds(o, L)]
                    half[pl.ds(o, L)] = x >> 1
                    par[pl.ds(o, L)] = (x & 1).astype(jnp.float32)
            return 0

        lax.fori_loop(0, NB // (2 * L), split, 0)

        def chunk(c, _):
            gw = pltpu.async_copy(
                w2_hbm.at[half_w.at[pl.ds(c * CH, CH)]], rows_w, sem_w)
            gv = pltpu.async_copy(
                v2_hbm.at[half_v.at[pl.ds(c * CH, CH)]], rows_v, sem_v)
            gn = pltpu.async_copy(
                v2_hbm.at[half_n.at[pl.ds(c * CH * NEG, CH * NEG)]],
                rows_n, sem_n)
            gw.wait()
            gv.wait()
            gn.wait()

            pw = par_w[pl.ds(c * CH, CH)]
            pv = par_v[pl.ds(c * CH, CH)]
            pn = [par_n[pl.ds(c * CH * NEG + t * L, L)] for t in range(NEG)]

            def pick(rows, r, k, p):
                a0 = rows[r, pl.ds(k * L, L)]
                a1 = rows[r, pl.ds(D + k * L, L)]
                return a0 + (a1 - a0) * p

            accp = jnp.zeros((16,), jnp.float32)
            accn = [jnp.zeros((16,), jnp.float32) for _ in range(NEG)]
            for b in range(CH):
                spw = jnp.take(pw, jnp.full((16,), b, jnp.int32))
                spv = jnp.take(pv, jnp.full((16,), b, jnp.int32))
                vv = [pick(rows_v, b, k, spv) for k in range(KD)]
                p = pick(rows_w, b, 0, spw) * vv[0]
                for k in range(1, KD):
                    p = p + pick(rows_w, b, k, spw) * vv[k]
                accp = jnp.where(lane == b, jnp.sum(p), accp)
                for n in range(NEG):
                    e = b * NEG + n
                    spn = jnp.take(pn[e // L], jnp.full((16,), e % L, jnp.int32))
                    q = pick(rows_n, e, 0, spn) * vv[0]
                    for k in range(1, KD):
                        q = q + pick(rows_n, e, k, spn) * vv[k]
                    accn[n] = jnp.where(lane == b, jnp.sum(q), accn[n])

            pos_buf[pl.ds(c * CH, CH)] = accp
            for n in range(NEG):
                neg_buf[pl.ds(n * NB + c * CH, CH)] = accn[n]
            return 0

        lax.fori_loop(0, NCHUNK, chunk, 0)

        pltpu.sync_copy(pos_buf, pos_out_hbm.at[pl.ds(base, NB)])
        pltpu.sync_copy(neg_buf, neg_out_hbm.at[pl.ds(nbase, NB * NEG)])

    return body


_SC_SCORES = _sc_scores()


def _tc_loss_body(p_ref, n_ref, o_ref):
    p = p_ref[...]
    n = -n_ref[...]
    lsp = jnp.minimum(p, 0.0) - jnp.log1p(jnp.exp(-jnp.abs(p)))
    lsn = jnp.minimum(n, 0.0) - jnp.log1p(jnp.exp(-jnp.abs(n)))
    o_ref[0, 0] = -(jnp.sum(lsp) + jnp.sum(lsn))


def kernel(pos_w, pos_v, neg_v, w_emb, v_emb):
    pos_w = pos_w.astype(jnp.int32)
    pos_v = pos_v.astype(jnp.int32)
    neg_flat = neg_v.reshape(-1).astype(jnp.int32)
    w2 = w_emb.reshape(V // 2, 2 * D)
    v2 = v_emb.reshape(V // 2, 2 * D)

    pos_raw, neg_raw = _SC_SCORES(pos_w, pos_v, neg_flat, w2, v2)

    loss = pl.pallas_call(
        _tc_loss_body,
        out_shape=jax.ShapeDtypeStruct((1, 1), jnp.float32),
        out_specs=pl.BlockSpec(memory_space=pltpu.SMEM),
    )(pos_raw.reshape(B // 128, 128), neg_raw.reshape(B * NEG // 128, 128))
    return loss[0, 0]


# speedup vs baseline: 8.5120x; 1.0741x over previous
import functools

import jax
import jax.numpy as jnp
from jax import lax
from jax.experimental import pallas as pl
from jax.experimental.pallas import tpu as pltpu
from jax.experimental.pallas import tpu_sc as plsc

B = 16384
V = 1000000
D = 64
DP = 128
NEG = 5

NC = 2
NS = 16
L = 16
NW = NC * NS
NB = B // NW
CH = 16
NCHUNK = NB // CH
KD = D // L


def _sc_scores():
    mesh = plsc.VectorSubcoreMesh(
        core_axis_name="c", subcore_axis_name="s", num_cores=NC, num_subcores=NS
    )

    @functools.partial(
        pl.kernel,
        mesh=mesh,
        compiler_params=pltpu.CompilerParams(needs_layout_passes=False),
        out_type=[
            jax.ShapeDtypeStruct((B,), jnp.float32),
            jax.ShapeDtypeStruct((B * NEG,), jnp.float32),
        ],
        scratch_types=[
            pltpu.VMEM((NB,), jnp.int32),
            pltpu.VMEM((NB,), jnp.int32),
            pltpu.VMEM((NB * NEG,), jnp.int32),
            pltpu.VMEM((CH, DP), jnp.float32),
            pltpu.VMEM((CH, DP), jnp.float32),
            pltpu.VMEM((CH * NEG, DP), jnp.float32),
            pltpu.VMEM((NB,), jnp.float32),
            pltpu.VMEM((NB * NEG,), jnp.float32),
            pltpu.SemaphoreType.DMA,
            pltpu.SemaphoreType.DMA,
            pltpu.SemaphoreType.DMA,
        ],
    )
    def body(pos_w_hbm, pos_v_hbm, neg_hbm, wp_hbm, vp_hbm,
             pos_out_hbm, neg_out_hbm,
             idx_w, idx_v, idx_n, rows_w, rows_v, rows_n,
             pos_buf, neg_buf, sem_w, sem_v, sem_n):
        wid = lax.axis_index("s") * NC + lax.axis_index("c")
        base = wid * NB
        nbase = wid * NB * NEG

        pltpu.sync_copy(pos_w_hbm.at[pl.ds(base, NB)], idx_w)
        pltpu.sync_copy(pos_v_hbm.at[pl.ds(base, NB)], idx_v)
        pltpu.sync_copy(neg_hbm.at[pl.ds(nbase, NB * NEG)], idx_n)

        lane = lax.iota(jnp.int32, 16)

        def chunk(c, _):
            gw = pltpu.async_copy(
                wp_hbm.at[idx_w.at[pl.ds(c * CH, CH)]], rows_w, sem_w)
            gv = pltpu.async_copy(
                vp_hbm.at[idx_v.at[pl.ds(c * CH, CH)]], rows_v, sem_v)
            gn = pltpu.async_copy(
                vp_hbm.at[idx_n.at[pl.ds(c * CH * NEG, CH * NEG)]],
                rows_n, sem_n)
            gw.wait()
            gv.wait()
            gn.wait()

            accp = jnp.zeros((16,), jnp.float32)
            accn = [jnp.zeros((16,), jnp.float32) for _ in range(NEG)]
            for b in range(CH):
                vv = [rows_v[b, pl.ds(k * L, L)] for k in range(KD)]
                p = rows_w[b, pl.ds(0, L)] * vv[0]
                for k in range(1, KD):
                    p = p + rows_w[b, pl.ds(k * L, L)] * vv[k]
                accp = jnp.where(lane == b, jnp.sum(p), accp)
                for n in range(NEG):
                    r = b * NEG + n
                    q = rows_n[r, pl.ds(0, L)] * vv[0]
                    for k in range(1, KD):
                        q = q + rows_n[r, pl.ds(k * L, L)] * vv[k]
                    accn[n] = jnp.where(lane == b, jnp.sum(q), accn[n])

            pos_buf[pl.ds(c * CH, CH)] = accp
            for n in range(NEG):
                neg_buf[pl.ds(n * NB + c * CH, CH)] = accn[n]
            return 0

        lax.fori_loop(0, NCHUNK, chunk, 0)

        pltpu.sync_copy(pos_buf, pos_out_hbm.at[pl.ds(base, NB)])
        pltpu.sync_copy(neg_buf, neg_out_hbm.at[pl.ds(nbase, NB * NEG)])

    return body


_SC_SCORES = _sc_scores()


def _tc_loss_body(p_ref, n_ref, o_ref):
    p = p_ref[...]
    n = -n_ref[...]
    lsp = jnp.minimum(p, 0.0) - jnp.log1p(jnp.exp(-jnp.abs(p)))
    lsn = jnp.minimum(n, 0.0) - jnp.log1p(jnp.exp(-jnp.abs(n)))
    o_ref[0, 0] = -(jnp.sum(lsp) + jnp.sum(lsn))


def kernel(pos_w, pos_v, neg_v, w_emb, v_emb):
    pos_w = pos_w.astype(jnp.int32)
    pos_v = pos_v.astype(jnp.int32)
    neg_flat = neg_v.reshape(-1).astype(jnp.int32)
    wp = jnp.pad(w_emb, ((0, 0), (0, DP - D)))
    vp = jnp.pad(v_emb, ((0, 0), (0, DP - D)))

    pos_raw, neg_raw = _SC_SCORES(pos_w, pos_v, neg_flat, wp, vp)

    loss = pl.pallas_call(
        _tc_loss_body,
        out_shape=jax.ShapeDtypeStruct((1, 1), jnp.float32),
        out_specs=pl.BlockSpec(memory_space=pltpu.SMEM),
    )(pos_raw.reshape(B // 128, 128), neg_raw.reshape(B * NEG // 128, 128))
    return loss[0, 0]


# speedup vs baseline: 10.6845x; 1.2552x over previous
import functools

import jax
import jax.numpy as jnp
from jax import lax
from jax.experimental import pallas as pl
from jax.experimental.pallas import tpu as pltpu
from jax.experimental.pallas import tpu_sc as plsc

B = 16384
V = 1000000
D = 64
DP = 128
NEG = 5

NC = 2
NS = 16
L = 16
NW = NC * NS
NB = B // NW
CH = 16
NCHUNK = NB // CH
KD = D // L


def _sc_scores():
    mesh = plsc.VectorSubcoreMesh(
        core_axis_name="c", subcore_axis_name="s", num_cores=NC, num_subcores=NS
    )

    @functools.partial(
        pl.kernel,
        mesh=mesh,
        compiler_params=pltpu.CompilerParams(needs_layout_passes=False),
        out_type=[
            jax.ShapeDtypeStruct((B,), jnp.float32),
            jax.ShapeDtypeStruct((B * NEG,), jnp.float32),
        ],
        scratch_types=[
            pltpu.VMEM((NB,), jnp.int32),
            pltpu.VMEM((NB,), jnp.int32),
            pltpu.VMEM((NB * NEG,), jnp.int32),
            pltpu.VMEM((CH, DP), jnp.float32),
            pltpu.VMEM((CH, DP), jnp.float32),
            pltpu.VMEM((CH * NEG, DP), jnp.float32),
            pltpu.VMEM((NB,), jnp.float32),
            pltpu.VMEM((NB * NEG,), jnp.float32),
            pltpu.SemaphoreType.DMA,
            pltpu.SemaphoreType.DMA,
            pltpu.SemaphoreType.DMA,
        ],
    )
    def body(pos_w_hbm, pos_v_hbm, neg_hbm, wp_hbm, vp_hbm,
             pos_out_hbm, neg_out_hbm,
             idx_w, idx_v, idx_n, rows_w, rows_v, rows_n,
             pos_buf, neg_buf, sem_w, sem_v, sem_n):
        wid = lax.axis_index("s") * NC + lax.axis_index("c")
        base = wid * NB
        nbase = wid * NB * NEG

        pltpu.sync_copy(pos_w_hbm.at[pl.ds(base, NB)], idx_w)
        pltpu.sync_copy(pos_v_hbm.at[pl.ds(base, NB)], idx_v)
        pltpu.sync_copy(neg_hbm.at[pl.ds(nbase, NB * NEG)], idx_n)

        lane = lax.iota(jnp.int32, 16)

        def chunk(c, _):
            gw = pltpu.async_copy(
                wp_hbm.at[idx_w.at[pl.ds(c * CH, CH)]], rows_w, sem_w)
            gv = pltpu.async_copy(
                vp_hbm.at[idx_v.at[pl.ds(c * CH, CH)]], rows_v, sem_v)
            gn = pltpu.async_copy(
                vp_hbm.at[idx_n.at[pl.ds(c * CH * NEG, CH * NEG)]],
                rows_n, sem_n)
            gw.wait()
            gv.wait()
            gn.wait()

            accp = jnp.zeros((16,), jnp.float32)
            accn = [jnp.zeros((16,), jnp.float32) for _ in range(NEG)]
            for b in range(CH):
                vv = [rows_v[b, pl.ds(k * L, L)] for k in range(KD)]
                p = rows_w[b, pl.ds(0, L)] * vv[0]
                for k in range(1, KD):
                    p = p + rows_w[b, pl.ds(k * L, L)] * vv[k]
                accp = jnp.where(lane == b, jnp.sum(p), accp)
                for n in range(NEG):
                    r = b * NEG + n
                    q = rows_n[r, pl.ds(0, L)] * vv[0]
                    for k in range(1, KD):
                        q = q + rows_n[r, pl.ds(k * L, L)] * vv[k]
                    accn[n] = jnp.where(lane == b, jnp.sum(q), accn[n])

            pos_buf[pl.ds(c * CH, CH)] = accp
            for n in range(NEG):
                neg_buf[pl.ds(n * NB + c * CH, CH)] = accn[n]
            return 0

        lax.fori_loop(0, NCHUNK, chunk, 0)

        pltpu.sync_copy(pos_buf, pos_out_hbm.at[pl.ds(base, NB)])
        pltpu.sync_copy(neg_buf, neg_out_hbm.at[pl.ds(nbase, NB * NEG)])

    return body


_SC_SCORES = _sc_scores()


def _tc_transpad_body(x_ref, o_ref):
    t = x_ref[...].T
    o_ref[:, :D] = t
    o_ref[:, D:] = jnp.zeros_like(t)


def _tc_transpad(tT):
    CB = 4096
    return pl.pallas_call(
        _tc_transpad_body,
        grid=(pl.cdiv(V, CB),),
        in_specs=[pl.BlockSpec((D, CB), lambda c: (0, c))],
        out_specs=pl.BlockSpec((CB, DP), lambda c: (c, 0)),
        out_shape=jax.ShapeDtypeStruct((V, DP), jnp.float32),
    )(tT)


def _tc_loss_body(p_ref, n_ref, o_ref):
    p = p_ref[...]
    n = -n_ref[...]
    lsp = jnp.minimum(p, 0.0) - jnp.log1p(jnp.exp(-jnp.abs(p)))
    lsn = jnp.minimum(n, 0.0) - jnp.log1p(jnp.exp(-jnp.abs(n)))
    o_ref[0, 0] = -(jnp.sum(lsp) + jnp.sum(lsn))


def kernel(pos_w, pos_v, neg_v, w_emb, v_emb):
    pos_w = pos_w.astype(jnp.int32)
    pos_v = pos_v.astype(jnp.int32)
    neg_flat = neg_v.reshape(-1).astype(jnp.int32)
    wp = _tc_transpad(w_emb.T)
    vp = jnp.pad(v_emb, ((0, 0), (0, DP - D)))

    pos_raw, neg_raw = _SC_SCORES(pos_w, pos_v, neg_flat, wp, vp)

    loss = pl.pallas_call(
        _tc_loss_body,
        out_shape=jax.ShapeDtypeStruct((1, 1), jnp.float32),
        out_specs=pl.BlockSpec(memory_space=pltpu.SMEM),
    )(pos_raw.reshape(B // 128, 128), neg_raw.reshape(B * NEG // 128, 128))
    return loss[0, 0]


# speedup vs baseline: 10.7075x; 1.0021x over previous
import functools

import jax
import jax.numpy as jnp
from jax import lax
from jax.experimental import pallas as pl
from jax.experimental.pallas import tpu as pltpu
from jax.experimental.pallas import tpu_sc as plsc

B = 16384
V = 1000000
D = 64
DP = 128
NEG = 5

NC = 2
NS = 16
L = 16
NW = NC * NS
NB = B // NW
CH = 16
NCHUNK = NB // CH
KD = D // L


def _sc_scores():
    mesh = plsc.VectorSubcoreMesh(
        core_axis_name="c", subcore_axis_name="s", num_cores=NC, num_subcores=NS
    )

    @functools.partial(
        pl.kernel,
        mesh=mesh,
        compiler_params=pltpu.CompilerParams(needs_layout_passes=False),
        out_type=[
            jax.ShapeDtypeStruct((B,), jnp.float32),
            jax.ShapeDtypeStruct((B * NEG,), jnp.float32),
        ],
        scratch_types=[
            pltpu.VMEM((NB,), jnp.int32),
            pltpu.VMEM((NB,), jnp.int32),
            pltpu.VMEM((NB * NEG,), jnp.int32),
            pltpu.VMEM((CH, DP), jnp.float32),
            pltpu.VMEM((CH, DP), jnp.float32),
            pltpu.VMEM((CH * NEG, DP), jnp.float32),
            pltpu.VMEM((NB,), jnp.float32),
            pltpu.VMEM((NB * NEG,), jnp.float32),
            pltpu.SemaphoreType.DMA,
            pltpu.SemaphoreType.DMA,
            pltpu.SemaphoreType.DMA,
        ],
    )
    def body(pos_w_hbm, pos_v_hbm, neg_hbm, wp_hbm, vp_hbm,
             pos_out_hbm, neg_out_hbm,
             idx_w, idx_v, idx_n, rows_w, rows_v, rows_n,
             pos_buf, neg_buf, sem_w, sem_v, sem_n):
        wid = lax.axis_index("s") * NC + lax.axis_index("c")
        base = wid * NB
        nbase = wid * NB * NEG

        pltpu.sync_copy(pos_w_hbm.at[pl.ds(base, NB)], idx_w)
        pltpu.sync_copy(pos_v_hbm.at[pl.ds(base, NB)], idx_v)
        pltpu.sync_copy(neg_hbm.at[pl.ds(nbase, NB * NEG)], idx_n)

        lane = lax.iota(jnp.int32, 16)

        def chunk(c, _):
            gw = pltpu.async_copy(
                wp_hbm.at[idx_w.at[pl.ds(c * CH, CH)]], rows_w, sem_w)
            gv = pltpu.async_copy(
                vp_hbm.at[idx_v.at[pl.ds(c * CH, CH)]], rows_v, sem_v)
            gn = pltpu.async_copy(
                vp_hbm.at[idx_n.at[pl.ds(c * CH * NEG, CH * NEG)]],
                rows_n, sem_n)
            gw.wait()
            gv.wait()
            gn.wait()

            accp = jnp.zeros((16,), jnp.float32)
            accn = [jnp.zeros((16,), jnp.float32) for _ in range(NEG)]
            for b in range(CH):
                vv = [rows_v[b, pl.ds(k * L, L)] for k in range(KD)]
                p = rows_w[b, pl.ds(0, L)] * vv[0]
                for k in range(1, KD):
                    p = p + rows_w[b, pl.ds(k * L, L)] * vv[k]
                accp = jnp.where(lane == b, jnp.sum(p), accp)
                for n in range(NEG):
                    r = b * NEG + n
                    q = rows_n[r, pl.ds(0, L)] * vv[0]
                    for k in range(1, KD):
                        q = q + rows_n[r, pl.ds(k * L, L)] * vv[k]
                    accn[n] = jnp.where(lane == b, jnp.sum(q), accn[n])

            pos_buf[pl.ds(c * CH, CH)] = accp
            for n in range(NEG):
                neg_buf[pl.ds(n * NB + c * CH, CH)] = accn[n]
            return 0

        lax.fori_loop(0, NCHUNK, chunk, 0)

        pltpu.sync_copy(pos_buf, pos_out_hbm.at[pl.ds(base, NB)])
        pltpu.sync_copy(neg_buf, neg_out_hbm.at[pl.ds(nbase, NB * NEG)])

    return body


_SC_SCORES = _sc_scores()


def _tc_transpad_body(x_ref, o_ref):
    t = x_ref[...].T
    o_ref[:, :D] = t
    o_ref[:, D:] = jnp.zeros_like(t)


def _tc_transpad(tT):
    CB = 4096
    return pl.pallas_call(
        _tc_transpad_body,
        grid=(pl.cdiv(V, CB),),
        in_specs=[pl.BlockSpec((D, CB), lambda c: (0, c))],
        out_specs=pl.BlockSpec((CB, DP), lambda c: (c, 0)),
        out_shape=jax.ShapeDtypeStruct((V, DP), jnp.float32),
    )(tT)


def _tc_loss_body(p_ref, n_ref, o_ref):
    p = p_ref[...]
    n = -n_ref[...]
    lsp = jnp.minimum(p, 0.0) - jnp.log1p(jnp.exp(-jnp.abs(p)))
    lsn = jnp.minimum(n, 0.0) - jnp.log1p(jnp.exp(-jnp.abs(n)))
    o_ref[0, 0] = -(jnp.sum(lsp) + jnp.sum(lsn))


def kernel(pos_w, pos_v, neg_v, w_emb, v_emb):
    pos_w = pos_w.astype(jnp.int32)
    pos_v = pos_v.astype(jnp.int32)
    neg_flat = neg_v.reshape(-1).astype(jnp.int32)
    wp = jnp.pad(w_emb, ((0, 0), (0, DP - D)))
    vp = _tc_transpad(v_emb.T)

    pos_raw, neg_raw = _SC_SCORES(pos_w, pos_v, neg_flat, wp, vp)

    loss = pl.pallas_call(
        _tc_loss_body,
        out_shape=jax.ShapeDtypeStruct((1, 1), jnp.float32),
        out_specs=pl.BlockSpec(memory_space=pltpu.SMEM),
    )(pos_raw.reshape(B // 128, 128), neg_raw.reshape(B * NEG // 128, 128))
    return loss[0, 0]
